# baseline (speedup 1.0000x reference)
import jax
import jax.numpy as jnp
from jax.experimental import pallas as pl
from jax.experimental.pallas import tpu as pltpu

_K = 4
_NEG = -3.0e38


def _matcher_kernel(iseg_ref, pseg_ref, isim_ref, psim_ref, out_ref):
    chunk = pl.program_id(1)

    x_i = isim_ref[0]
    x_p = psim_ref[0]
    w_i = iseg_ref[0]
    w_p = pseg_ref[0]

    def global_ch(c):
        r = x_i * w_i[c, :][:, None]
        return jnp.max(r, axis=0)

    v1 = jnp.max(x_p, axis=1, keepdims=True)
    m2 = jnp.where(x_p < v1, x_p, _NEG)
    v2 = jnp.max(m2, axis=1, keepdims=True)
    m3 = jnp.where(m2 < v2, m2, _NEG)
    v3 = jnp.max(m3, axis=1, keepdims=True)
    m4 = jnp.where(m3 < v3, m3, _NEG)
    v4 = jnp.max(m4, axis=1, keepdims=True)
    c1 = jnp.sum((x_p >= v1).astype(jnp.float32), axis=1, keepdims=True)
    c2 = jnp.sum((x_p >= v2).astype(jnp.float32), axis=1, keepdims=True)
    c3 = jnp.sum((x_p >= v3).astype(jnp.float32), axis=1, keepdims=True)
    cut = jnp.where(
        c1 >= _K, v1, jnp.where(c2 >= _K, v2, jnp.where(c3 >= _K, v3, v4))
    )
    mn = jnp.min(x_p, axis=1, keepdims=True)
    masked = jnp.where(x_p < cut, _NEG, x_p)

    def local_ch(c):
        w = w_p[c, :][:, None]
        keep = jnp.max(masked * w, axis=0)
        mn_term = jnp.max(mn * w)
        return jnp.maximum(keep, mn_term)

    part = jnp.stack(
        [global_ch(0), global_ch(1), local_ch(0), local_ch(1)], axis=0
    )

    @pl.when(chunk == 0)
    def _init():
        out_ref[0] = part

    @pl.when(chunk != 0)
    def _acc():
        out_ref[0] = jnp.maximum(out_ref[0], part)


def kernel(init_sim, prev_sim, init_seg, prev_seg):
    B, HW, H, W = init_sim.shape
    QL = H * W
    M_BLK = 512
    n_chunks = HW // M_BLK

    isim = init_sim.reshape(B, HW, QL)
    psim = prev_sim.reshape(B, HW, QL)
    iseg = init_seg.reshape(B, 2, HW)
    pseg = prev_seg.reshape(B, 2, HW)

    out = pl.pallas_call(
        _matcher_kernel,
        grid=(B, n_chunks),
        in_specs=[
            pl.BlockSpec((1, 2, M_BLK), lambda b, c: (b, 0, c)),
            pl.BlockSpec((1, 2, M_BLK), lambda b, c: (b, 0, c)),
            pl.BlockSpec((1, M_BLK, QL), lambda b, c: (b, c, 0)),
            pl.BlockSpec((1, M_BLK, QL), lambda b, c: (b, c, 0)),
        ],
        out_specs=pl.BlockSpec((1, 4, QL), lambda b, c: (b, 0, 0)),
        out_shape=jax.ShapeDtypeStruct((B, 4, QL), jnp.float32),
        compiler_params=pltpu.CompilerParams(
            dimension_semantics=("parallel", "arbitrary"),
        ),
    )(iseg, pseg, isim, psim)

    return out.reshape(B, 4, H, W)


# speedup vs baseline: 1.0311x; 1.0311x over previous
import jax
import jax.numpy as jnp
from jax.experimental import pallas as pl
from jax.experimental.pallas import tpu as pltpu

_K = 4
_NEG = -3.0e38


def _matcher_kernel(iseg_ref, pseg_ref, isim_ref, psim_ref, out_ref):
    chunk = pl.program_id(1)

    x_i = isim_ref[0]
    x_p = psim_ref[0]
    w_i = iseg_ref[0]
    w_p = pseg_ref[0]

    def global_ch(c):
        r = x_i * w_i[c, :][:, None]
        return jnp.max(r, axis=0)

    hw = x_p.shape[1]
    lanes = 128
    ncol = hw // lanes
    a1 = x_p[:, 0:lanes]
    neg_a = jnp.full_like(a1, _NEG)
    a2 = neg_a
    a3 = neg_a
    a4 = neg_a
    amin = a1
    for k in range(1, ncol):
        s = x_p[:, k * lanes : (k + 1) * lanes]
        amin = jnp.minimum(amin, s)
        t = s
        n = jnp.maximum(a1, t)
        t = jnp.minimum(a1, t)
        a1 = n
        n = jnp.maximum(a2, t)
        t = jnp.minimum(a2, t)
        a2 = n
        n = jnp.maximum(a3, t)
        t = jnp.minimum(a3, t)
        a3 = n
        a4 = jnp.maximum(a4, t)
    cand = jnp.concatenate([a1, a2, a3, a4], axis=1)

    v1 = jnp.max(cand, axis=1, keepdims=True)
    m2 = jnp.where(cand < v1, cand, _NEG)
    v2 = jnp.max(m2, axis=1, keepdims=True)
    m3 = jnp.where(m2 < v2, m2, _NEG)
    v3 = jnp.max(m3, axis=1, keepdims=True)
    m4 = jnp.where(m3 < v3, m3, _NEG)
    v4 = jnp.max(m4, axis=1, keepdims=True)
    c1 = jnp.sum((cand >= v1).astype(jnp.float32), axis=1, keepdims=True)
    c2 = jnp.sum((cand >= v2).astype(jnp.float32), axis=1, keepdims=True)
    c3 = jnp.sum((cand >= v3).astype(jnp.float32), axis=1, keepdims=True)
    cut = jnp.where(
        c1 >= _K, v1, jnp.where(c2 >= _K, v2, jnp.where(c3 >= _K, v3, v4))
    )
    mn = jnp.min(amin, axis=1, keepdims=True)
    masked = jnp.where(x_p < cut, _NEG, x_p)

    def local_ch(c):
        w = w_p[c, :][:, None]
        keep = jnp.max(masked * w, axis=0)
        mn_term = jnp.max(mn * w)
        return jnp.maximum(keep, mn_term)

    part = jnp.stack(
        [global_ch(0), global_ch(1), local_ch(0), local_ch(1)], axis=0
    )

    @pl.when(chunk == 0)
    def _init():
        out_ref[0] = part

    @pl.when(chunk != 0)
    def _acc():
        out_ref[0] = jnp.maximum(out_ref[0], part)


def kernel(init_sim, prev_sim, init_seg, prev_seg):
    B, HW, H, W = init_sim.shape
    QL = H * W
    M_BLK = 512
    n_chunks = HW // M_BLK

    isim = init_sim.reshape(B, HW, QL)
    psim = prev_sim.reshape(B, HW, QL)
    iseg = init_seg.reshape(B, 2, HW)
    pseg = prev_seg.reshape(B, 2, HW)

    out = pl.pallas_call(
        _matcher_kernel,
        grid=(B, n_chunks),
        in_specs=[
            pl.BlockSpec((1, 2, M_BLK), lambda b, c: (b, 0, c)),
            pl.BlockSpec((1, 2, M_BLK), lambda b, c: (b, 0, c)),
            pl.BlockSpec((1, M_BLK, QL), lambda b, c: (b, c, 0)),
            pl.BlockSpec((1, M_BLK, QL), lambda b, c: (b, c, 0)),
        ],
        out_specs=pl.BlockSpec((1, 4, QL), lambda b, c: (b, 0, 0)),
        out_shape=jax.ShapeDtypeStruct((B, 4, QL), jnp.float32),
        compiler_params=pltpu.CompilerParams(
            dimension_semantics=("parallel", "arbitrary"),
        ),
    )(iseg, pseg, isim, psim)

    return out.reshape(B, 4, H, W)


# speedup vs baseline: 1.0491x; 1.0175x over previous
import jax
import jax.numpy as jnp
from jax.experimental import pallas as pl
from jax.experimental.pallas import tpu as pltpu

_K = 4
_NEG = -3.0e38


def _matcher_kernel(iseg_ref, pseg_ref, isim_ref, psim_ref, out_ref):
    chunk = pl.program_id(1)

    x_i = isim_ref[0]
    x_p = psim_ref[0]
    w_i = iseg_ref[0]
    w_p = pseg_ref[0]

    def global_ch(c):
        r = x_i * w_i[c, :][:, None]
        return jnp.max(r, axis=0)

    hw = x_p.shape[1]
    lanes = 128
    ncol = hw // lanes
    a1 = x_p[:, 0:lanes]
    neg_a = jnp.full_like(a1, _NEG)
    a2 = neg_a
    a3 = neg_a
    a4 = neg_a
    amin = a1
    for k in range(1, ncol):
        s = x_p[:, k * lanes : (k + 1) * lanes]
        amin = jnp.minimum(amin, s)
        t = s
        n = jnp.maximum(a1, t)
        t = jnp.minimum(a1, t)
        a1 = n
        n = jnp.maximum(a2, t)
        t = jnp.minimum(a2, t)
        a2 = n
        n = jnp.maximum(a3, t)
        t = jnp.minimum(a3, t)
        a3 = n
        a4 = jnp.maximum(a4, t)
    cand = jnp.concatenate([a1, a2, a3, a4], axis=1)

    v1 = jnp.max(cand, axis=1, keepdims=True)
    m2 = jnp.where(cand < v1, cand, _NEG)
    v2 = jnp.max(m2, axis=1, keepdims=True)
    m3 = jnp.where(m2 < v2, m2, _NEG)
    v3 = jnp.max(m3, axis=1, keepdims=True)
    m4 = jnp.where(m3 < v3, m3, _NEG)
    v4 = jnp.max(m4, axis=1, keepdims=True)
    c1 = jnp.sum((cand >= v1).astype(jnp.float32), axis=1, keepdims=True)
    c2 = jnp.sum((cand >= v2).astype(jnp.float32), axis=1, keepdims=True)
    c3 = jnp.sum((cand >= v3).astype(jnp.float32), axis=1, keepdims=True)
    cut = jnp.where(
        c1 >= _K, v1, jnp.where(c2 >= _K, v2, jnp.where(c3 >= _K, v3, v4))
    )
    mn = jnp.min(amin, axis=1, keepdims=True)
    masked = jnp.where(x_p < cut, _NEG, x_p)

    def local_ch(c):
        w = w_p[c, :][:, None]
        keep = jnp.max(masked * w, axis=0)
        mn_term = jnp.max(mn * w)
        return jnp.maximum(keep, mn_term)

    part = jnp.stack(
        [global_ch(0), global_ch(1), local_ch(0), local_ch(1)], axis=0
    )

    @pl.when(chunk == 0)
    def _init():
        out_ref[0] = part

    @pl.when(chunk != 0)
    def _acc():
        out_ref[0] = jnp.maximum(out_ref[0], part)


def kernel(init_sim, prev_sim, init_seg, prev_seg):
    B, HW, H, W = init_sim.shape
    QL = H * W
    M_BLK = 1024
    n_chunks = HW // M_BLK

    isim = init_sim.reshape(B, HW, QL)
    psim = prev_sim.reshape(B, HW, QL)
    iseg = init_seg.reshape(B, 2, HW)
    pseg = prev_seg.reshape(B, 2, HW)

    out = pl.pallas_call(
        _matcher_kernel,
        grid=(B, n_chunks),
        in_specs=[
            pl.BlockSpec((1, 2, M_BLK), lambda b, c: (b, 0, c)),
            pl.BlockSpec((1, 2, M_BLK), lambda b, c: (b, 0, c)),
            pl.BlockSpec((1, M_BLK, QL), lambda b, c: (b, c, 0)),
            pl.BlockSpec((1, M_BLK, QL), lambda b, c: (b, c, 0)),
        ],
        out_specs=pl.BlockSpec((1, 4, QL), lambda b, c: (b, 0, 0)),
        out_shape=jax.ShapeDtypeStruct((B, 4, QL), jnp.float32),
        compiler_params=pltpu.CompilerParams(
            dimension_semantics=("parallel", "arbitrary"),
        ),
    )(iseg, pseg, isim, psim)

    return out.reshape(B, 4, H, W)
